# Initial kernel scaffold; baseline (speedup 1.0000x reference)
import jax
import jax.numpy as jnp
from jax.experimental import pallas as pl


def kernel(dq1, dq2, dp1, dp2, m, t, dt, length, k, Wn_enc, bn_enc, We_enc, be_enc, W_msg, b_msg, W_upd, b_upd, W_dec, b_dec):
    raise NotImplementedError("write your pallas kernel here")


# speedup vs baseline: 104.1148x; 104.1148x over previous
import jax
import jax.numpy as jnp
from jax.experimental import pallas as pl
from jax.experimental.pallas import tpu as pltpu

B, P, D, H = 32, 64, 2, 64


def _body(na_ref, ea_ref, wn_ref, bn_ref, we_ref, be_ref,
          wme_ref, wms_ref, wmr_ref, bm_ref, wu1_ref, wu2_ref, bu_ref,
          wd_ref, bd_ref, out_ref):
    f32 = jnp.float32
    hn = jax.nn.relu(jnp.dot(na_ref[0], wn_ref[...],
                             preferred_element_type=f32) + bn_ref[...])
    a_term = jnp.dot(hn, wms_ref[...], preferred_element_type=f32)
    c_term = jnp.dot(hn, wmr_ref[...], preferred_element_type=f32)
    he = jax.nn.relu(jnp.dot(ea_ref[0], we_ref[...],
                             preferred_element_type=f32) + be_ref[...])
    eh = jnp.dot(he, wme_ref[...], preferred_element_type=f32)
    eh3 = eh.reshape(P, P, H)
    bm = bm_ref[...]
    aggs = []
    for d in range(D):
        a_d = a_term[d * P:(d + 1) * P]
        c_d = c_term[d * P:(d + 1) * P]
        t = jax.nn.relu(eh3 + a_d[None, :, :] + c_d[:, None, :] + bm)
        aggs.append(jnp.sum(t, axis=1))
    agg = jnp.concatenate(aggs, axis=0)
    h2 = jax.nn.relu(jnp.dot(hn, wu1_ref[...], preferred_element_type=f32)
                     + jnp.dot(agg, wu2_ref[...], preferred_element_type=f32)
                     + bu_ref[...])
    out_ref[0] = jnp.dot(h2, wd_ref[...], preferred_element_type=f32) + bd_ref[...]


def kernel(dq1, dq2, dp1, dp2, m, t, dt, length, k,
           Wn_enc, bn_enc, We_enc, be_enc, W_msg, b_msg, W_upd, b_upd,
           W_dec, b_dec):
    del t, dt
    m_rep = jnp.tile(m, (1, 1, D))
    na = jnp.stack([dq1, dq2, dp1, dp2, m_rep], axis=-1)
    na = na.transpose(0, 2, 1, 3).reshape(B, D * P, 5)
    ea = jnp.stack([length, k], axis=-1).reshape(B, P * P, 2)
    wme, wms, wmr = W_msg[:H], W_msg[H:2 * H], W_msg[2 * H:]
    wu1, wu2 = W_upd[:H], W_upd[H:]
    row = lambda v: v.reshape(1, -1)

    per_b3 = lambda shape: pl.BlockSpec(shape, lambda b: (b, 0, 0))
    const2 = lambda shape: pl.BlockSpec(shape, lambda b: (0, 0))

    out = pl.pallas_call(
        _body,
        grid=(B,),
        in_specs=[
            per_b3((1, D * P, 5)),
            per_b3((1, P * P, 2)),
            const2((5, H)),
            const2((1, H)),
            const2((2, H)),
            const2((1, H)),
            const2((H, H)),
            const2((H, H)),
            const2((H, H)),
            const2((1, H)),
            const2((H, H)),
            const2((H, H)),
            const2((1, H)),
            const2((H, 2)),
            const2((1, 2)),
        ],
        out_specs=per_b3((1, D * P, 2)),
        out_shape=jax.ShapeDtypeStruct((B, D * P, 2), jnp.float32),
        compiler_params=pltpu.CompilerParams(
            dimension_semantics=("arbitrary",)),
    )(na, ea, Wn_enc, row(bn_enc), We_enc, row(be_enc),
      wme, wms, wmr, row(b_msg), wu1, wu2, row(b_upd), W_dec, row(b_dec))

    r = out.reshape(B, D, P, 2)
    return r[..., 0].transpose(0, 2, 1), r[..., 1].transpose(0, 2, 1)


# speedup vs baseline: 112.6322x; 1.0818x over previous
import jax
import jax.numpy as jnp
from jax.experimental import pallas as pl
from jax.experimental.pallas import tpu as pltpu

B, P, D, H = 32, 64, 2, 64
G = 1


def _body(na_ref, ea_ref, wn_ref, bn_ref, we_ref, be_ref,
          wme_ref, wms_ref, wmr_ref, bm_ref, wu1_ref, wu2_ref, bu_ref,
          wd_ref, bd_ref, out_ref):
    f32 = jnp.float32
    for g in range(G):
        hn = jax.nn.relu(jnp.dot(na_ref[g], wn_ref[...],
                                 preferred_element_type=f32) + bn_ref[...])
        a_term = jnp.dot(hn, wms_ref[...], preferred_element_type=f32)
        c_term = jnp.dot(hn, wmr_ref[...], preferred_element_type=f32)
        he = jax.nn.relu(jnp.dot(ea_ref[g], we_ref[...],
                                 preferred_element_type=f32) + be_ref[...])
        eh = jnp.dot(he, wme_ref[...], preferred_element_type=f32)
        eh3 = eh.reshape(P, P, 2 * H)
        t = jax.nn.relu(eh3 + a_term[None, :, :] + c_term[:, None, :]
                        + bm_ref[...])
        agg = jnp.sum(t, axis=1)
        h2 = jax.nn.relu(jnp.dot(hn, wu1_ref[...], preferred_element_type=f32)
                         + jnp.dot(agg, wu2_ref[...], preferred_element_type=f32)
                         + bu_ref[...])
        out_ref[g] = jnp.dot(h2, wd_ref[...],
                             preferred_element_type=f32) + bd_ref[...]


def _blkdiag(w):
    r, c = w.shape
    z = jnp.zeros((r, c), w.dtype)
    return jnp.concatenate(
        [jnp.concatenate([w, z], axis=1), jnp.concatenate([z, w], axis=1)],
        axis=0)


def kernel(dq1, dq2, dp1, dp2, m, t, dt, length, k,
           Wn_enc, bn_enc, We_enc, be_enc, W_msg, b_msg, W_upd, b_upd,
           W_dec, b_dec):
    del t, dt
    m_rep = jnp.tile(m, (1, 1, D))
    na = jnp.stack([dq1, dq2, dp1, dp2, m_rep], axis=-1).reshape(B, P, D * 5)
    ea = jnp.stack([length, k], axis=-1).reshape(B, P * P, 2)
    wme, wms, wmr = W_msg[:H], W_msg[H:2 * H], W_msg[2 * H:]
    wu1, wu2 = W_upd[:H], W_upd[H:]
    two = lambda v: jnp.concatenate([v, v]).reshape(1, -1)

    per_b3 = lambda shape: pl.BlockSpec(shape, lambda b: (b, 0, 0))
    const2 = lambda shape: pl.BlockSpec(shape, lambda b: (0, 0))

    out = pl.pallas_call(
        _body,
        grid=(B // G,),
        in_specs=[
            per_b3((G, P, D * 5)),
            per_b3((G, P * P, 2)),
            const2((D * 5, D * H)),
            const2((1, D * H)),
            const2((2, H)),
            const2((1, H)),
            const2((H, D * H)),
            const2((D * H, D * H)),
            const2((D * H, D * H)),
            const2((1, D * H)),
            const2((D * H, D * H)),
            const2((D * H, D * H)),
            const2((1, D * H)),
            const2((D * H, D * 2)),
            const2((1, D * 2)),
        ],
        out_specs=per_b3((G, P, D * 2)),
        out_shape=jax.ShapeDtypeStruct((B, P, D * 2), jnp.float32),
        compiler_params=pltpu.CompilerParams(
            dimension_semantics=("arbitrary",)),
    )(na, ea, _blkdiag(Wn_enc), two(bn_enc), We_enc, be_enc.reshape(1, -1),
      jnp.concatenate([wme, wme], axis=1), _blkdiag(wms), _blkdiag(wmr),
      two(b_msg), _blkdiag(wu1), _blkdiag(wu2), two(b_upd),
      _blkdiag(W_dec), two(b_dec))

    r = out.reshape(B, P, D, 2)
    return r[..., 0], r[..., 1]


# speedup vs baseline: 122.1777x; 1.0847x over previous
import jax
import jax.numpy as jnp
from jax.experimental import pallas as pl
from jax.experimental.pallas import tpu as pltpu

B, P, D, H = 32, 64, 2, 64
G = 4


def _body(na_ref, ea_ref, wn_ref, bn_ref, we_ref, be_ref,
          wme_ref, wms_ref, wmr_ref, bm_ref, wu1_ref, wu2_ref, bu_ref,
          wd_ref, bd_ref, out_ref):
    f32 = jnp.float32
    for g in range(G):
        hn = jax.nn.relu(jnp.dot(na_ref[g], wn_ref[...],
                                 preferred_element_type=f32) + bn_ref[...])
        a_term = jnp.dot(hn, wms_ref[...], preferred_element_type=f32)
        c_term = jnp.dot(hn, wmr_ref[...], preferred_element_type=f32)
        he = jax.nn.relu(jnp.dot(ea_ref[g], we_ref[...],
                                 preferred_element_type=f32) + be_ref[...])
        eh = jnp.dot(he, wme_ref[...], preferred_element_type=f32)
        eh3 = eh.reshape(P, P, 2 * H)
        t = jax.nn.relu(eh3 + a_term[None, :, :] + c_term[:, None, :]
                        + bm_ref[...])
        agg = jnp.sum(t, axis=1)
        h2 = jax.nn.relu(jnp.dot(hn, wu1_ref[...], preferred_element_type=f32)
                         + jnp.dot(agg, wu2_ref[...], preferred_element_type=f32)
                         + bu_ref[...])
        out_ref[g] = jnp.dot(h2, wd_ref[...],
                             preferred_element_type=f32) + bd_ref[...]


def _blkdiag(w):
    r, c = w.shape
    z = jnp.zeros((r, c), w.dtype)
    return jnp.concatenate(
        [jnp.concatenate([w, z], axis=1), jnp.concatenate([z, w], axis=1)],
        axis=0)


def kernel(dq1, dq2, dp1, dp2, m, t, dt, length, k,
           Wn_enc, bn_enc, We_enc, be_enc, W_msg, b_msg, W_upd, b_upd,
           W_dec, b_dec):
    del t, dt
    m_rep = jnp.tile(m, (1, 1, D))
    na = jnp.stack([dq1, dq2, dp1, dp2, m_rep], axis=-1).reshape(B, P, D * 5)
    ea = jnp.stack([length, k], axis=-1).reshape(B, P * P, 2)
    wme, wms, wmr = W_msg[:H], W_msg[H:2 * H], W_msg[2 * H:]
    wu1, wu2 = W_upd[:H], W_upd[H:]
    two = lambda v: jnp.concatenate([v, v]).reshape(1, -1)

    per_b3 = lambda shape: pl.BlockSpec(shape, lambda b: (b, 0, 0))
    const2 = lambda shape: pl.BlockSpec(shape, lambda b: (0, 0))

    out = pl.pallas_call(
        _body,
        grid=(B // G,),
        in_specs=[
            per_b3((G, P, D * 5)),
            per_b3((G, P * P, 2)),
            const2((D * 5, D * H)),
            const2((1, D * H)),
            const2((2, H)),
            const2((1, H)),
            const2((H, D * H)),
            const2((D * H, D * H)),
            const2((D * H, D * H)),
            const2((1, D * H)),
            const2((D * H, D * H)),
            const2((D * H, D * H)),
            const2((1, D * H)),
            const2((D * H, D * 2)),
            const2((1, D * 2)),
        ],
        out_specs=per_b3((G, P, D * 2)),
        out_shape=jax.ShapeDtypeStruct((B, P, D * 2), jnp.float32),
        compiler_params=pltpu.CompilerParams(
            dimension_semantics=("arbitrary",)),
    )(na, ea, _blkdiag(Wn_enc), two(bn_enc), We_enc, be_enc.reshape(1, -1),
      jnp.concatenate([wme, wme], axis=1), _blkdiag(wms), _blkdiag(wmr),
      two(b_msg), _blkdiag(wu1), _blkdiag(wu2), two(b_upd),
      _blkdiag(W_dec), two(b_dec))

    r = out.reshape(B, P, D, 2)
    return r[..., 0], r[..., 1]


# speedup vs baseline: 135.2887x; 1.1073x over previous
import jax
import jax.numpy as jnp
from jax.experimental import pallas as pl
from jax.experimental.pallas import tpu as pltpu

B, P, D, H = 32, 64, 2, 64
G = 4


def _body(na_ref, ea_ref, wn_ref, bn_ref, we_ref, be_ref,
          wme_ref, wms_ref, wmr_ref, bm_ref, wu1_ref, wu2_ref, bu_ref,
          wd_ref, bd_ref, out_ref):
    f32 = jnp.float32
    na = na_ref[...].reshape(G * P, D * 5)
    hn = jax.nn.relu(jnp.dot(na, wn_ref[...],
                             preferred_element_type=f32) + bn_ref[...])
    a_term = jnp.dot(hn, wms_ref[...], preferred_element_type=f32)
    c_term = jnp.dot(hn, wmr_ref[...], preferred_element_type=f32)
    he = jax.nn.relu(jnp.dot(ea_ref[...].reshape(G * P * P, 2), we_ref[...],
                             preferred_element_type=f32) + be_ref[...])
    eh = jnp.dot(he, wme_ref[...], preferred_element_type=f32)
    eh4 = eh.reshape(G, P, P, D * H)
    t = jax.nn.relu(eh4 + a_term.reshape(G, 1, P, D * H)
                    + c_term.reshape(G, P, 1, D * H) + bm_ref[...])
    agg = jnp.sum(t, axis=2).reshape(G * P, D * H)
    h2 = jax.nn.relu(jnp.dot(hn, wu1_ref[...], preferred_element_type=f32)
                     + jnp.dot(agg, wu2_ref[...], preferred_element_type=f32)
                     + bu_ref[...])
    o = jnp.dot(h2, wd_ref[...], preferred_element_type=f32) + bd_ref[...]
    out_ref[...] = o.reshape(G, P, D * 2)


def _blkdiag(w):
    r, c = w.shape
    z = jnp.zeros((r, c), w.dtype)
    return jnp.concatenate(
        [jnp.concatenate([w, z], axis=1), jnp.concatenate([z, w], axis=1)],
        axis=0)


def kernel(dq1, dq2, dp1, dp2, m, t, dt, length, k,
           Wn_enc, bn_enc, We_enc, be_enc, W_msg, b_msg, W_upd, b_upd,
           W_dec, b_dec):
    del t, dt
    m_rep = jnp.tile(m, (1, 1, D))
    na = jnp.stack([dq1, dq2, dp1, dp2, m_rep], axis=-1).reshape(B, P, D * 5)
    ea = jnp.stack([length, k], axis=-1).reshape(B, P * P, 2)
    wme, wms, wmr = W_msg[:H], W_msg[H:2 * H], W_msg[2 * H:]
    wu1, wu2 = W_upd[:H], W_upd[H:]
    two = lambda v: jnp.concatenate([v, v]).reshape(1, -1)

    per_b3 = lambda shape: pl.BlockSpec(shape, lambda b: (b, 0, 0))
    const2 = lambda shape: pl.BlockSpec(shape, lambda b: (0, 0))

    out = pl.pallas_call(
        _body,
        grid=(B // G,),
        in_specs=[
            per_b3((G, P, D * 5)),
            per_b3((G, P * P, 2)),
            const2((D * 5, D * H)),
            const2((1, D * H)),
            const2((2, H)),
            const2((1, H)),
            const2((H, D * H)),
            const2((D * H, D * H)),
            const2((D * H, D * H)),
            const2((1, D * H)),
            const2((D * H, D * H)),
            const2((D * H, D * H)),
            const2((1, D * H)),
            const2((D * H, D * 2)),
            const2((1, D * 2)),
        ],
        out_specs=per_b3((G, P, D * 2)),
        out_shape=jax.ShapeDtypeStruct((B, P, D * 2), jnp.float32),
        compiler_params=pltpu.CompilerParams(
            dimension_semantics=("arbitrary",)),
    )(na, ea, _blkdiag(Wn_enc), two(bn_enc), We_enc, be_enc.reshape(1, -1),
      jnp.concatenate([wme, wme], axis=1), _blkdiag(wms), _blkdiag(wmr),
      two(b_msg), _blkdiag(wu1), _blkdiag(wu2), two(b_upd),
      _blkdiag(W_dec), two(b_dec))

    r = out.reshape(B, P, D, 2)
    return r[..., 0], r[..., 1]
